# Initial kernel scaffold; baseline (speedup 1.0000x reference)
import jax
import jax.numpy as jnp
from jax.experimental import pallas as pl


def kernel(x, edge_index, edge_attr, batch, W_node, b_node, W_edge, b_edge, Wsrc, bsrc, Wdst, bdst, Wm1, bm1, Wm2, bm2, ln_g, ln_b, Wh1, bh1, Wh2, bh2, Wg1, bg1, Wg2, bg2):
    raise NotImplementedError("write your pallas kernel here")


# speedup vs baseline: 5.8139x; 5.8139x over previous
import functools

import jax
import jax.numpy as jnp
from jax import lax
from jax.experimental import pallas as pl
from jax.experimental.pallas import tpu as pltpu
from jax.experimental.pallas import tpu_sc as plsc

N = 10000
E = 320000
NODE_IN = 128
EDGE_IN = 16
H = 64
L = 4
NG = 16

NC = 2
NS = 16
NW = NC * NS
EPW = E // NW
K = 80
CPT = EPW // K
NP = 10240
RPS = NP // NS
ZR = 128
DW = 16

_mesh = plsc.VectorSubcoreMesh(
    core_axis_name="c", subcore_axis_name="s", num_cores=NC, num_subcores=NS)


def _edge_body(with_deg, *refs):
    if with_deg:
        (a_hbm, b_hbm, c_hbm, src_hbm, dst_hbm, s_out, d_out,
         sidx, didx, buf_a, buf_b, buf_c, buf_o, ones_b, zb, zb16,
         s_sh, d_sh, sem_a, sem_b, sem_c) = refs
    else:
        (a_hbm, b_hbm, c_hbm, src_hbm, dst_hbm, s_out,
         sidx, didx, buf_a, buf_b, buf_c, buf_o, zb,
         s_sh, sem_a, sem_b, sem_c) = refs
    ci = lax.axis_index("c")
    si = lax.axis_index("s")
    wid = si * NC + ci

    zero16 = jnp.zeros((16,), jnp.float32)
    def zrow(i, _):
        for q in range(H // 16):
            zb[i, pl.ds(q * 16, 16)] = zero16
        return 0
    lax.fori_loop(0, ZR, zrow, 0)
    def zcopy(j, _):
        pltpu.sync_copy(zb, s_sh.at[pl.ds(si * RPS + j * ZR, ZR)])
        return 0
    lax.fori_loop(0, RPS // ZR, zcopy, 0)
    if with_deg:
        def zrow16(i, _):
            zb16[i, :] = zero16
            return 0
        lax.fori_loop(0, ZR, zrow16, 0)
        def zcopy16(j, _):
            pltpu.sync_copy(zb16, d_sh.at[pl.ds(si * RPS + j * ZR, ZR)])
            return 0
        lax.fori_loop(0, RPS // ZR, zcopy16, 0)
        one16 = jnp.ones((16,), jnp.float32)
        def orow(i, _):
            ones_b[i, :] = one16
            return 0
        lax.fori_loop(0, K, orow, 0)
    plsc.subcore_barrier()

    pltpu.sync_copy(src_hbm.at[wid], sidx)
    pltpu.sync_copy(dst_hbm.at[wid], didx)

    def chunk(i, _):
        svec = sidx.at[i]
        dvec = didx.at[i]
        cp_a = pltpu.async_copy(a_hbm.at[svec], buf_a, sem_a)
        cp_b = pltpu.async_copy(b_hbm.at[dvec], buf_b, sem_b)
        cp_c = pltpu.async_copy(c_hbm.at[pl.ds(wid * EPW + i * K, K)], buf_c,
                                sem_c)
        cp_a.wait()
        cp_b.wait()
        cp_c.wait()
        def row(r, _):
            for q in range(H // 16):
                sl = pl.ds(q * 16, 16)
                buf_o[r, sl] = jnp.maximum(
                    buf_a[r, sl] + buf_b[r, sl] + buf_c[r, sl], 0.0)
            return 0
        lax.fori_loop(0, K, row, 0)
        pltpu.sync_copy(buf_o, s_sh.at[dvec], add=True)
        if with_deg:
            pltpu.sync_copy(ones_b, d_sh.at[dvec], add=True)
        return 0
    lax.fori_loop(0, CPT, chunk, 0)

    plsc.subcore_barrier()
    def wcopy(j, _):
        rows = pl.ds(si * RPS + j * ZR, ZR)
        pltpu.sync_copy(s_sh.at[rows], s_out.at[ci, rows])
        if with_deg:
            pltpu.sync_copy(d_sh.at[rows], d_out.at[ci, rows])
        return 0
    lax.fori_loop(0, RPS // ZR, wcopy, 0)


def _make_edge_kernel(with_deg):
    out_type = [jax.ShapeDtypeStruct((NC, NP, H), jnp.float32)]
    scratch = [
        pltpu.VMEM((CPT, K), jnp.int32),
        pltpu.VMEM((CPT, K), jnp.int32),
        pltpu.VMEM((K, H), jnp.float32),
        pltpu.VMEM((K, H), jnp.float32),
        pltpu.VMEM((K, H), jnp.float32),
        pltpu.VMEM((K, H), jnp.float32),
    ]
    if with_deg:
        out_type.append(jax.ShapeDtypeStruct((NC, NP, DW), jnp.float32))
        scratch.append(pltpu.VMEM((K, DW), jnp.float32))
    scratch.append(pltpu.VMEM((ZR, H), jnp.float32))
    if with_deg:
        scratch.append(pltpu.VMEM((ZR, DW), jnp.float32))
    scratch.append(pltpu.VMEM_SHARED((NP, H), jnp.float32))
    if with_deg:
        scratch.append(pltpu.VMEM_SHARED((NP, DW), jnp.float32))
    scratch += [pltpu.SemaphoreType.DMA] * 3
    return pl.kernel(
        functools.partial(_edge_body, with_deg),
        out_type=tuple(out_type) if with_deg else out_type[0],
        mesh=_mesh,
        scratch_types=scratch,
        compiler_params=pltpu.CompilerParams(use_tc_tiling_on_sc=False),
    )


_edge_deg = _make_edge_kernel(True)
_edge = _make_edge_kernel(False)


EB = 8000


def _edgeprep_body(ea_ref, wc_ref, cc_ref, out_ref):
    ea = ea_ref[...]
    for l in range(L):
        out_ref[l] = (jnp.dot(ea, wc_ref[l], preferred_element_type=jnp.float32)
                      + cc_ref[l])


_edgeprep = pl.pallas_call(
    _edgeprep_body,
    grid=(E // EB,),
    in_specs=[
        pl.BlockSpec((EB, EDGE_IN), lambda i: (i, 0)),
        pl.BlockSpec((L, EDGE_IN, H), lambda i: (0, 0, 0)),
        pl.BlockSpec((L, H), lambda i: (0, 0)),
    ],
    out_specs=pl.BlockSpec((L, EB, H), lambda i: (0, i, 0)),
    out_shape=jax.ShapeDtypeStruct((L, E, H), jnp.float32),
)


def _h0_body(x_ref, wn_ref, bn_ref, wa_ref, ca_ref, wb_ref, cb_ref,
             h_ref, a_ref, b_ref):
    h = jnp.dot(x_ref[...], wn_ref[...],
                preferred_element_type=jnp.float32) + bn_ref[...]
    h_ref[...] = h
    a_ref[...] = jnp.dot(h, wa_ref[...],
                         preferred_element_type=jnp.float32) + ca_ref[...]
    b_ref[...] = jnp.dot(h, wb_ref[...],
                         preferred_element_type=jnp.float32) + cb_ref[...]


_h0 = pl.pallas_call(
    _h0_body,
    out_shape=(jax.ShapeDtypeStruct((N, H), jnp.float32),
               jax.ShapeDtypeStruct((N, H), jnp.float32),
               jax.ShapeDtypeStruct((N, H), jnp.float32)),
)


def _post_body(last, *refs):
    if last:
        (h_ref, a_ref, b_ref, s_ref, deg_ref, bm1_ref, wm2_ref, bm2_ref,
         hn_ref) = refs
    else:
        (h_ref, a_ref, b_ref, s_ref, deg_ref, bm1_ref, wm2_ref, bm2_ref,
         wa_ref, ca_ref, wb_ref, cb_ref, hn_ref, an_ref, bn_ref) = refs
    s = s_ref[0, :N, :] + s_ref[1, :N, :] + jnp.maximum(
        a_ref[...] + b_ref[...] + bm1_ref[...], 0.0)
    degf = deg_ref[0, :N, 0:1] + deg_ref[1, :N, 0:1] + 1.0
    aggr = jnp.dot(s, wm2_ref[...],
                   preferred_element_type=jnp.float32) + degf * bm2_ref[...]
    hn = jnp.maximum(h_ref[...] + aggr, 0.0)
    hn_ref[...] = hn
    if not last:
        an_ref[...] = jnp.dot(hn, wa_ref[...],
                              preferred_element_type=jnp.float32) + ca_ref[...]
        bn_ref[...] = jnp.dot(hn, wb_ref[...],
                              preferred_element_type=jnp.float32) + cb_ref[...]


_post_mid = pl.pallas_call(
    functools.partial(_post_body, False),
    out_shape=(jax.ShapeDtypeStruct((N, H), jnp.float32),
               jax.ShapeDtypeStruct((N, H), jnp.float32),
               jax.ShapeDtypeStruct((N, H), jnp.float32)),
)

_post_last = pl.pallas_call(
    functools.partial(_post_body, True),
    out_shape=jax.ShapeDtypeStruct((N, H), jnp.float32),
)


def _final_body(h_ref, batch_ref, lng_ref, lnb_ref, wh1_ref, bh1_ref,
                wh2_ref, bh2_ref, wg1_ref, bg1_ref, wg2_ref, bg2_ref,
                nl_ref, pat_ref):
    h = h_ref[...]
    mu = jnp.mean(h, axis=1, keepdims=True)
    var = jnp.mean((h - mu) ** 2, axis=1, keepdims=True)
    xn = (h - mu) * lax.rsqrt(var + 1e-5) * lng_ref[...] + lnb_ref[...]
    t = jnp.maximum(
        jnp.dot(xn, wh1_ref[...], preferred_element_type=jnp.float32)
        + bh1_ref[...], 0.0)
    logit = jnp.dot(t, wh2_ref[...],
                    preferred_element_type=jnp.float32) + bh2_ref[...]
    nl_ref[...] = jax.nn.sigmoid(logit)
    m = (batch_ref[...] == lax.broadcasted_iota(jnp.int32, (1, NG), 1)
         ).astype(jnp.float32)
    sums = lax.dot_general(m, h, (((0,), (0,)), ((), ())),
                           preferred_element_type=jnp.float32)
    counts = lax.dot_general(m, jnp.ones((N, 1), jnp.float32),
                             (((0,), (0,)), ((), ())),
                             preferred_element_type=jnp.float32)
    g = sums / jnp.maximum(counts, 1.0)
    gt = jnp.maximum(
        jnp.dot(g, wg1_ref[...], preferred_element_type=jnp.float32)
        + bg1_ref[...], 0.0)
    pat_ref[...] = jnp.dot(gt, wg2_ref[...],
                           preferred_element_type=jnp.float32) + bg2_ref[...]


_final = pl.pallas_call(
    _final_body,
    out_shape=(jax.ShapeDtypeStruct((N, 1), jnp.float32),
               jax.ShapeDtypeStruct((NG, 8), jnp.float32)),
)


def kernel(x, edge_index, edge_attr, batch, W_node, b_node, W_edge, b_edge,
           Wsrc, bsrc, Wdst, bdst, Wm1, bm1, Wm2, bm2, ln_g, ln_b,
           Wh1, bh1, Wh2, bh2, Wg1, bg1, Wg2, bg2):
    f32 = jnp.float32
    Wm1s, Wm1d, Wm1e = Wm1[:, :H, :], Wm1[:, H:2 * H, :], Wm1[:, 2 * H:, :]
    WA = jnp.einsum('lij,ljk->lik', Wsrc, Wm1s)
    cA = jnp.einsum('lj,ljk->lk', bsrc, Wm1s)
    WB = jnp.einsum('lij,ljk->lik', Wdst, Wm1d)
    cB = jnp.einsum('lj,ljk->lk', bdst, Wm1d)
    WC = jnp.einsum('ij,ljk->lik', W_edge, Wm1e)
    cC = jnp.einsum('j,ljk->lk', b_edge, Wm1e) + bm1

    c_all = _edgeprep(edge_attr, WC, cC)
    srcs = edge_index[0].reshape(NW, CPT, K)
    dsts = edge_index[1].reshape(NW, CPT, K)

    h, a, b = _h0(x, W_node, b_node.reshape(1, H),
                  WA[0], cA[0].reshape(1, H), WB[0], cB[0].reshape(1, H))

    deg_raw = None
    for l in range(L):
        if l == 0:
            s_raw, deg_raw = _edge_deg(a, b, c_all[l], srcs, dsts)
        else:
            s_raw = _edge(a, b, c_all[l], srcs, dsts)
        if l < L - 1:
            h, a, b = _post_mid(
                h, a, b, s_raw, deg_raw, bm1[l].reshape(1, H), Wm2[l],
                bm2[l].reshape(1, H),
                WA[l + 1], cA[l + 1].reshape(1, H),
                WB[l + 1], cB[l + 1].reshape(1, H))
        else:
            h = _post_last(
                h, a, b, s_raw, deg_raw, bm1[l].reshape(1, H), Wm2[l],
                bm2[l].reshape(1, H))

    wg2p = jnp.concatenate([Wg2, jnp.zeros((32, 3), f32)], axis=1)
    bg2p = jnp.concatenate([bg2, jnp.zeros((3,), f32)]).reshape(1, 8)
    nl, pat = _final(
        h, batch.reshape(N, 1), ln_g.reshape(1, H), ln_b.reshape(1, H),
        Wh1, bh1.reshape(1, 32), Wh2, bh2.reshape(1, 1),
        Wg1, bg1.reshape(1, 32), wg2p, bg2p)
    return nl.reshape(-1), pat[:, :5]


# speedup vs baseline: 7.1513x; 1.2300x over previous
import functools

import jax
import jax.numpy as jnp
from jax import lax
from jax.experimental import pallas as pl
from jax.experimental.pallas import tpu as pltpu
from jax.experimental.pallas import tpu_sc as plsc

N = 10000
E = 320000
NODE_IN = 128
EDGE_IN = 16
H = 64
L = 4
NG = 16

NC = 2
NS = 16
NW = NC * NS
EPW = E // NW
K = 80
CPT = EPW // K
NP = 10240
RPS = NP // NS
ZR = 128
DW = 16

_mesh = plsc.VectorSubcoreMesh(
    core_axis_name="c", subcore_axis_name="s", num_cores=NC, num_subcores=NS)


def _edge_body(with_deg, *refs):
    if with_deg:
        (a_hbm, b_hbm, c_hbm, src_hbm, dst_hbm, s_out, d_out,
         sidx, didx, buf_a, buf_b, buf_c, buf_o, ones_b, zb, zb16,
         s_sh, d_sh, sem_a, sem_b, sem_c) = refs
    else:
        (a_hbm, b_hbm, c_hbm, src_hbm, dst_hbm, s_out,
         sidx, didx, buf_a, buf_b, buf_c, buf_o, zb,
         s_sh, sem_a, sem_b, sem_c) = refs
    ci = lax.axis_index("c")
    si = lax.axis_index("s")
    wid = si * NC + ci

    zero16 = jnp.zeros((16,), jnp.float32)
    def zrow(i, _):
        for q in range(H // 16):
            zb[i, pl.ds(q * 16, 16)] = zero16
        return 0
    lax.fori_loop(0, ZR, zrow, 0)
    def zcopy(j, _):
        pltpu.sync_copy(zb, s_sh.at[pl.ds(si * RPS + j * ZR, ZR)])
        return 0
    lax.fori_loop(0, RPS // ZR, zcopy, 0)
    if with_deg:
        def zrow16(i, _):
            zb16[i, :] = zero16
            return 0
        lax.fori_loop(0, ZR, zrow16, 0)
        def zcopy16(j, _):
            pltpu.sync_copy(zb16, d_sh.at[pl.ds(si * RPS + j * ZR, ZR)])
            return 0
        lax.fori_loop(0, RPS // ZR, zcopy16, 0)
        one16 = jnp.ones((16,), jnp.float32)
        def orow(i, _):
            ones_b[i, :] = one16
            return 0
        lax.fori_loop(0, K, orow, 0)
    plsc.subcore_barrier()

    pltpu.sync_copy(src_hbm.at[wid], sidx)
    pltpu.sync_copy(dst_hbm.at[wid], didx)

    def chunk(i, _):
        svec = sidx.at[i]
        dvec = didx.at[i]
        cp_a = pltpu.async_copy(a_hbm.at[svec], buf_a, sem_a)
        cp_b = pltpu.async_copy(b_hbm.at[dvec], buf_b, sem_b)
        cp_c = pltpu.async_copy(c_hbm.at[pl.ds(wid * EPW + i * K, K)], buf_c,
                                sem_c)
        cp_a.wait()
        cp_b.wait()
        cp_c.wait()
        def row(r, _):
            for q in range(H // 16):
                sl = pl.ds(q * 16, 16)
                buf_o[r, sl] = jnp.maximum(
                    buf_a[r, sl] + buf_b[r, sl] + buf_c[r, sl], 0.0)
            return 0
        lax.fori_loop(0, K, row, 0)
        pltpu.sync_copy(buf_o, s_sh.at[dvec], add=True)
        if with_deg:
            pltpu.sync_copy(ones_b, d_sh.at[dvec], add=True)
        return 0
    lax.fori_loop(0, CPT, chunk, 0)

    plsc.subcore_barrier()
    def wcopy(j, _):
        rows = pl.ds(si * RPS + j * ZR, ZR)
        pltpu.sync_copy(s_sh.at[rows], s_out.at[ci, rows])
        if with_deg:
            pltpu.sync_copy(d_sh.at[rows], d_out.at[ci, rows])
        return 0
    lax.fori_loop(0, RPS // ZR, wcopy, 0)


def _make_edge_kernel(with_deg):
    out_type = [jax.ShapeDtypeStruct((NC, NP, H), jnp.float32)]
    scratch = [
        pltpu.VMEM((CPT, K), jnp.int32),
        pltpu.VMEM((CPT, K), jnp.int32),
        pltpu.VMEM((K, H), jnp.float32),
        pltpu.VMEM((K, H), jnp.float32),
        pltpu.VMEM((K, H), jnp.float32),
        pltpu.VMEM((K, H), jnp.float32),
    ]
    if with_deg:
        out_type.append(jax.ShapeDtypeStruct((NC, NP, DW), jnp.float32))
        scratch.append(pltpu.VMEM((K, DW), jnp.float32))
    scratch.append(pltpu.VMEM((ZR, H), jnp.float32))
    if with_deg:
        scratch.append(pltpu.VMEM((ZR, DW), jnp.float32))
    scratch.append(pltpu.VMEM_SHARED((NP, H), jnp.float32))
    if with_deg:
        scratch.append(pltpu.VMEM_SHARED((NP, DW), jnp.float32))
    scratch += [pltpu.SemaphoreType.DMA] * 3
    return pl.kernel(
        functools.partial(_edge_body, with_deg),
        out_type=tuple(out_type) if with_deg else out_type[0],
        mesh=_mesh,
        scratch_types=scratch,
        compiler_params=pltpu.CompilerParams(use_tc_tiling_on_sc=False),
    )


_edge_deg = _make_edge_kernel(True)
_edge = _make_edge_kernel(False)


EB = 8000


def _edgeprep_body(ea_ref, wc_ref, cc_ref, out_ref):
    out_ref[...] = (jnp.dot(ea_ref[...], wc_ref[...],
                            preferred_element_type=jnp.float32) + cc_ref[...])


_edgeprep = pl.pallas_call(
    _edgeprep_body,
    grid=(E // EB,),
    in_specs=[
        pl.BlockSpec((EB, EDGE_IN), lambda i: (i, 0)),
        pl.BlockSpec((EDGE_IN, H), lambda i: (0, 0)),
        pl.BlockSpec((1, H), lambda i: (0, 0)),
    ],
    out_specs=pl.BlockSpec((EB, H), lambda i: (i, 0)),
    out_shape=jax.ShapeDtypeStruct((E, H), jnp.float32),
)


def _h0_body(x_ref, wn_ref, bn_ref, wa_ref, ca_ref, wb_ref, cb_ref,
             h_ref, a_ref, b_ref):
    h = jnp.dot(x_ref[...], wn_ref[...],
                preferred_element_type=jnp.float32) + bn_ref[...]
    h_ref[...] = h
    a_ref[...] = jnp.dot(h, wa_ref[...],
                         preferred_element_type=jnp.float32) + ca_ref[...]
    b_ref[...] = jnp.dot(h, wb_ref[...],
                         preferred_element_type=jnp.float32) + cb_ref[...]


_h0 = pl.pallas_call(
    _h0_body,
    out_shape=(jax.ShapeDtypeStruct((N, H), jnp.float32),
               jax.ShapeDtypeStruct((N, H), jnp.float32),
               jax.ShapeDtypeStruct((N, H), jnp.float32)),
)


def _post_body(last, *refs):
    if last:
        (h_ref, a_ref, b_ref, s_ref, deg_ref, bm1_ref, wm2_ref, bm2_ref,
         hn_ref) = refs
    else:
        (h_ref, a_ref, b_ref, s_ref, deg_ref, bm1_ref, wm2_ref, bm2_ref,
         wa_ref, ca_ref, wb_ref, cb_ref, hn_ref, an_ref, bn_ref) = refs
    s = s_ref[0, :N, :] + s_ref[1, :N, :] + jnp.maximum(
        a_ref[...] + b_ref[...] + bm1_ref[...], 0.0)
    degf = deg_ref[0, :N, 0:1] + deg_ref[1, :N, 0:1] + 1.0
    aggr = jnp.dot(s, wm2_ref[...],
                   preferred_element_type=jnp.float32) + degf * bm2_ref[...]
    hn = jnp.maximum(h_ref[...] + aggr, 0.0)
    hn_ref[...] = hn
    if not last:
        an_ref[...] = jnp.dot(hn, wa_ref[...],
                              preferred_element_type=jnp.float32) + ca_ref[...]
        bn_ref[...] = jnp.dot(hn, wb_ref[...],
                              preferred_element_type=jnp.float32) + cb_ref[...]


_post_mid = pl.pallas_call(
    functools.partial(_post_body, False),
    out_shape=(jax.ShapeDtypeStruct((N, H), jnp.float32),
               jax.ShapeDtypeStruct((N, H), jnp.float32),
               jax.ShapeDtypeStruct((N, H), jnp.float32)),
)

_post_last = pl.pallas_call(
    functools.partial(_post_body, True),
    out_shape=jax.ShapeDtypeStruct((N, H), jnp.float32),
)


def _final_body(h_ref, batch_ref, lng_ref, lnb_ref, wh1_ref, bh1_ref,
                wh2_ref, bh2_ref, wg1_ref, bg1_ref, wg2_ref, bg2_ref,
                nl_ref, pat_ref):
    h = h_ref[...]
    mu = jnp.mean(h, axis=1, keepdims=True)
    var = jnp.mean((h - mu) ** 2, axis=1, keepdims=True)
    xn = (h - mu) * lax.rsqrt(var + 1e-5) * lng_ref[...] + lnb_ref[...]
    t = jnp.maximum(
        jnp.dot(xn, wh1_ref[...], preferred_element_type=jnp.float32)
        + bh1_ref[...], 0.0)
    logit = jnp.dot(t, wh2_ref[...],
                    preferred_element_type=jnp.float32) + bh2_ref[...]
    nl_ref[...] = jax.nn.sigmoid(logit)
    m = (batch_ref[...] == lax.broadcasted_iota(jnp.int32, (1, NG), 1)
         ).astype(jnp.float32)
    sums = lax.dot_general(m, h, (((0,), (0,)), ((), ())),
                           preferred_element_type=jnp.float32)
    counts = lax.dot_general(m, jnp.ones((N, 1), jnp.float32),
                             (((0,), (0,)), ((), ())),
                             preferred_element_type=jnp.float32)
    g = sums / jnp.maximum(counts, 1.0)
    gt = jnp.maximum(
        jnp.dot(g, wg1_ref[...], preferred_element_type=jnp.float32)
        + bg1_ref[...], 0.0)
    pat_ref[...] = jnp.dot(gt, wg2_ref[...],
                           preferred_element_type=jnp.float32) + bg2_ref[...]


_final = pl.pallas_call(
    _final_body,
    out_shape=(jax.ShapeDtypeStruct((N, 1), jnp.float32),
               jax.ShapeDtypeStruct((NG, 8), jnp.float32)),
)


def kernel(x, edge_index, edge_attr, batch, W_node, b_node, W_edge, b_edge,
           Wsrc, bsrc, Wdst, bdst, Wm1, bm1, Wm2, bm2, ln_g, ln_b,
           Wh1, bh1, Wh2, bh2, Wg1, bg1, Wg2, bg2):
    f32 = jnp.float32
    hi = jax.lax.Precision.HIGHEST
    Wm1s, Wm1d, Wm1e = Wm1[:, :H, :], Wm1[:, H:2 * H, :], Wm1[:, 2 * H:, :]
    WA = jnp.einsum('lij,ljk->lik', Wsrc, Wm1s, precision=hi)
    cA = jnp.einsum('lj,ljk->lk', bsrc, Wm1s, precision=hi)
    WB = jnp.einsum('lij,ljk->lik', Wdst, Wm1d, precision=hi)
    cB = jnp.einsum('lj,ljk->lk', bdst, Wm1d, precision=hi)
    WC = jnp.einsum('ij,ljk->lik', W_edge, Wm1e, precision=hi)
    cC = jnp.einsum('j,ljk->lk', b_edge, Wm1e, precision=hi) + bm1

    c_layers = [_edgeprep(edge_attr, WC[l], cC[l].reshape(1, H))
                for l in range(L)]
    srcs = edge_index[0].reshape(NW, CPT, K)
    dsts = edge_index[1].reshape(NW, CPT, K)

    h, a, b = _h0(x, W_node, b_node.reshape(1, H),
                  WA[0], cA[0].reshape(1, H), WB[0], cB[0].reshape(1, H))

    deg_raw = None
    for l in range(L):
        if l == 0:
            s_raw, deg_raw = _edge_deg(a, b, c_layers[l], srcs, dsts)
        else:
            s_raw = _edge(a, b, c_layers[l], srcs, dsts)
        if l < L - 1:
            h, a, b = _post_mid(
                h, a, b, s_raw, deg_raw, bm1[l].reshape(1, H), Wm2[l],
                bm2[l].reshape(1, H),
                WA[l + 1], cA[l + 1].reshape(1, H),
                WB[l + 1], cB[l + 1].reshape(1, H))
        else:
            h = _post_last(
                h, a, b, s_raw, deg_raw, bm1[l].reshape(1, H), Wm2[l],
                bm2[l].reshape(1, H))

    wg2p = jnp.concatenate([Wg2, jnp.zeros((32, 3), f32)], axis=1)
    bg2p = jnp.concatenate([bg2, jnp.zeros((3,), f32)]).reshape(1, 8)
    nl, pat = _final(
        h, batch.reshape(N, 1), ln_g.reshape(1, H), ln_b.reshape(1, H),
        Wh1, bh1.reshape(1, 32), Wh2, bh2.reshape(1, 1),
        Wg1, bg1.reshape(1, 32), wg2p, bg2p)
    return nl.reshape(-1), pat[:, :5]


# speedup vs baseline: 8.1557x; 1.1405x over previous
import functools

import jax
import jax.numpy as jnp
from jax import lax
from jax.experimental import pallas as pl
from jax.experimental.pallas import tpu as pltpu
from jax.experimental.pallas import tpu_sc as plsc

N = 10000
E = 320000
NODE_IN = 128
EDGE_IN = 16
H = 64
L = 4
NG = 16

NC = 2
NS = 16
NW = NC * NS
EPW = E // NW
K = 80
CPT = EPW // K
NP = 10240
RPS = NP // NS
ZR = 128
DW = 16

_mesh = plsc.VectorSubcoreMesh(
    core_axis_name="c", subcore_axis_name="s", num_cores=NC, num_subcores=NS)


def _edge_body(with_deg, *refs):
    if with_deg:
        (a_hbm, b_hbm, c_hbm, src_hbm, dst_hbm, s_out, d_out,
         sidx, didx, buf_a0, buf_b0, buf_c0, buf_o0,
         buf_a1, buf_b1, buf_c1, buf_o1, ones_b, zb, zb16,
         s_sh, d_sh, sem_a0, sem_b0, sem_c0, sem_a1, sem_b1, sem_c1) = refs
    else:
        (a_hbm, b_hbm, c_hbm, src_hbm, dst_hbm, s_out,
         sidx, didx, buf_a0, buf_b0, buf_c0, buf_o0,
         buf_a1, buf_b1, buf_c1, buf_o1, zb,
         s_sh, sem_a0, sem_b0, sem_c0, sem_a1, sem_b1, sem_c1) = refs
    ci = lax.axis_index("c")
    si = lax.axis_index("s")
    wid = si * NC + ci

    zero16 = jnp.zeros((16,), jnp.float32)
    def zrow(i, _):
        for q in range(H // 16):
            zb[i, pl.ds(q * 16, 16)] = zero16
        return 0
    lax.fori_loop(0, ZR, zrow, 0)
    def zcopy(j, _):
        pltpu.sync_copy(zb, s_sh.at[pl.ds(si * RPS + j * ZR, ZR)])
        return 0
    lax.fori_loop(0, RPS // ZR, zcopy, 0)
    if with_deg:
        def zrow16(i, _):
            zb16[i, :] = zero16
            return 0
        lax.fori_loop(0, ZR, zrow16, 0)
        def zcopy16(j, _):
            pltpu.sync_copy(zb16, d_sh.at[pl.ds(si * RPS + j * ZR, ZR)])
            return 0
        lax.fori_loop(0, RPS // ZR, zcopy16, 0)
        one16 = jnp.ones((16,), jnp.float32)
        def orow(i, _):
            ones_b[i, :] = one16
            return 0
        lax.fori_loop(0, K, orow, 0)
    plsc.subcore_barrier()

    def issue(c, buf_a, buf_b, buf_c, sem_a, sem_b, sem_c):
        pltpu.async_copy(a_hbm.at[sidx.at[c]], buf_a, sem_a)
        pltpu.async_copy(b_hbm.at[didx.at[c]], buf_b, sem_b)
        pltpu.async_copy(c_hbm.at[pl.ds(wid * EPW + c * K, K)], buf_c, sem_c)

    def half(c, buf_a, buf_b, buf_c, buf_o, sem_a, sem_b, sem_c):
        pltpu.make_async_copy(a_hbm.at[sidx.at[c]], buf_a, sem_a).wait()
        pltpu.make_async_copy(b_hbm.at[didx.at[c]], buf_b, sem_b).wait()
        pltpu.make_async_copy(c_hbm.at[pl.ds(wid * EPW + c * K, K)], buf_c,
                              sem_c).wait()
        def row(r, _):
            for q in range(H // 16):
                sl = pl.ds(q * 16, 16)
                buf_o[r, sl] = jnp.maximum(
                    buf_a[r, sl] + buf_b[r, sl] + buf_c[r, sl], 0.0)
            return 0
        lax.fori_loop(0, K, row, 0)
        @pl.when(c + 2 < CPT)
        def _():
            issue(c + 2, buf_a, buf_b, buf_c, sem_a, sem_b, sem_c)
        pltpu.sync_copy(buf_o, s_sh.at[didx.at[c]], add=True)
        if with_deg:
            pltpu.sync_copy(ones_b, d_sh.at[didx.at[c]], add=True)

    pltpu.sync_copy(src_hbm.at[wid], sidx)
    pltpu.sync_copy(dst_hbm.at[wid], didx)
    issue(0, buf_a0, buf_b0, buf_c0, sem_a0, sem_b0, sem_c0)
    issue(1, buf_a1, buf_b1, buf_c1, sem_a1, sem_b1, sem_c1)

    def pair(j, _):
        half(2 * j, buf_a0, buf_b0, buf_c0, buf_o0, sem_a0, sem_b0, sem_c0)
        half(2 * j + 1, buf_a1, buf_b1, buf_c1, buf_o1,
             sem_a1, sem_b1, sem_c1)
        return 0
    lax.fori_loop(0, (CPT - 1) // 2, pair, 0)
    half(CPT - 1, buf_a0, buf_b0, buf_c0, buf_o0, sem_a0, sem_b0, sem_c0)

    plsc.subcore_barrier()
    def wcopy(j, _):
        rows = pl.ds(si * RPS + j * ZR, ZR)
        pltpu.sync_copy(s_sh.at[rows], s_out.at[ci, rows])
        if with_deg:
            pltpu.sync_copy(d_sh.at[rows], d_out.at[ci, rows])
        return 0
    lax.fori_loop(0, RPS // ZR, wcopy, 0)


def _make_edge_kernel(with_deg):
    out_type = [jax.ShapeDtypeStruct((NC, NP, H), jnp.float32)]
    scratch = [
        pltpu.VMEM((CPT, K), jnp.int32),
        pltpu.VMEM((CPT, K), jnp.int32),
    ]
    scratch += [pltpu.VMEM((K, H), jnp.float32)] * 8
    if with_deg:
        out_type.append(jax.ShapeDtypeStruct((NC, NP, DW), jnp.float32))
        scratch.append(pltpu.VMEM((K, DW), jnp.float32))
    scratch.append(pltpu.VMEM((ZR, H), jnp.float32))
    if with_deg:
        scratch.append(pltpu.VMEM((ZR, DW), jnp.float32))
    scratch.append(pltpu.VMEM_SHARED((NP, H), jnp.float32))
    if with_deg:
        scratch.append(pltpu.VMEM_SHARED((NP, DW), jnp.float32))
    scratch += [pltpu.SemaphoreType.DMA] * 6
    return pl.kernel(
        functools.partial(_edge_body, with_deg),
        out_type=tuple(out_type) if with_deg else out_type[0],
        mesh=_mesh,
        scratch_types=scratch,
        compiler_params=pltpu.CompilerParams(use_tc_tiling_on_sc=False),
    )


_edge_deg = _make_edge_kernel(True)
_edge = _make_edge_kernel(False)


EB = 8000


def _edgeprep_body(ea_ref, wc_ref, cc_ref, out_ref):
    out_ref[...] = (jnp.dot(ea_ref[...], wc_ref[...],
                            preferred_element_type=jnp.float32) + cc_ref[...])


_edgeprep = pl.pallas_call(
    _edgeprep_body,
    grid=(E // EB,),
    in_specs=[
        pl.BlockSpec((EB, EDGE_IN), lambda i: (i, 0)),
        pl.BlockSpec((EDGE_IN, H), lambda i: (0, 0)),
        pl.BlockSpec((1, H), lambda i: (0, 0)),
    ],
    out_specs=pl.BlockSpec((EB, H), lambda i: (i, 0)),
    out_shape=jax.ShapeDtypeStruct((E, H), jnp.float32),
)


def _h0_body(x_ref, wn_ref, bn_ref, wa_ref, ca_ref, wb_ref, cb_ref,
             h_ref, a_ref, b_ref):
    h = jnp.dot(x_ref[...], wn_ref[...],
                preferred_element_type=jnp.float32) + bn_ref[...]
    h_ref[...] = h
    a_ref[...] = jnp.dot(h, wa_ref[...],
                         preferred_element_type=jnp.float32) + ca_ref[...]
    b_ref[...] = jnp.dot(h, wb_ref[...],
                         preferred_element_type=jnp.float32) + cb_ref[...]


_h0 = pl.pallas_call(
    _h0_body,
    out_shape=(jax.ShapeDtypeStruct((N, H), jnp.float32),
               jax.ShapeDtypeStruct((N, H), jnp.float32),
               jax.ShapeDtypeStruct((N, H), jnp.float32)),
)


def _post_body(last, *refs):
    if last:
        (h_ref, a_ref, b_ref, s_ref, deg_ref, bm1_ref, wm2_ref, bm2_ref,
         hn_ref) = refs
    else:
        (h_ref, a_ref, b_ref, s_ref, deg_ref, bm1_ref, wm2_ref, bm2_ref,
         wa_ref, ca_ref, wb_ref, cb_ref, hn_ref, an_ref, bn_ref) = refs
    s = s_ref[0, :N, :] + s_ref[1, :N, :] + jnp.maximum(
        a_ref[...] + b_ref[...] + bm1_ref[...], 0.0)
    degf = deg_ref[0, :N, 0:1] + deg_ref[1, :N, 0:1] + 1.0
    aggr = jnp.dot(s, wm2_ref[...],
                   preferred_element_type=jnp.float32) + degf * bm2_ref[...]
    hn = jnp.maximum(h_ref[...] + aggr, 0.0)
    hn_ref[...] = hn
    if not last:
        an_ref[...] = jnp.dot(hn, wa_ref[...],
                              preferred_element_type=jnp.float32) + ca_ref[...]
        bn_ref[...] = jnp.dot(hn, wb_ref[...],
                              preferred_element_type=jnp.float32) + cb_ref[...]


_post_mid = pl.pallas_call(
    functools.partial(_post_body, False),
    out_shape=(jax.ShapeDtypeStruct((N, H), jnp.float32),
               jax.ShapeDtypeStruct((N, H), jnp.float32),
               jax.ShapeDtypeStruct((N, H), jnp.float32)),
)

_post_last = pl.pallas_call(
    functools.partial(_post_body, True),
    out_shape=jax.ShapeDtypeStruct((N, H), jnp.float32),
)


def _final_body(h_ref, batch_ref, lng_ref, lnb_ref, wh1_ref, bh1_ref,
                wh2_ref, bh2_ref, wg1_ref, bg1_ref, wg2_ref, bg2_ref,
                nl_ref, pat_ref):
    h = h_ref[...]
    mu = jnp.mean(h, axis=1, keepdims=True)
    var = jnp.mean((h - mu) ** 2, axis=1, keepdims=True)
    xn = (h - mu) * lax.rsqrt(var + 1e-5) * lng_ref[...] + lnb_ref[...]
    t = jnp.maximum(
        jnp.dot(xn, wh1_ref[...], preferred_element_type=jnp.float32)
        + bh1_ref[...], 0.0)
    logit = jnp.dot(t, wh2_ref[...],
                    preferred_element_type=jnp.float32) + bh2_ref[...]
    nl_ref[...] = jax.nn.sigmoid(logit)
    m = (batch_ref[...] == lax.broadcasted_iota(jnp.int32, (1, NG), 1)
         ).astype(jnp.float32)
    sums = lax.dot_general(m, h, (((0,), (0,)), ((), ())),
                           preferred_element_type=jnp.float32)
    counts = lax.dot_general(m, jnp.ones((N, 1), jnp.float32),
                             (((0,), (0,)), ((), ())),
                             preferred_element_type=jnp.float32)
    g = sums / jnp.maximum(counts, 1.0)
    gt = jnp.maximum(
        jnp.dot(g, wg1_ref[...], preferred_element_type=jnp.float32)
        + bg1_ref[...], 0.0)
    pat_ref[...] = jnp.dot(gt, wg2_ref[...],
                           preferred_element_type=jnp.float32) + bg2_ref[...]


_final = pl.pallas_call(
    _final_body,
    out_shape=(jax.ShapeDtypeStruct((N, 1), jnp.float32),
               jax.ShapeDtypeStruct((NG, 8), jnp.float32)),
)


def kernel(x, edge_index, edge_attr, batch, W_node, b_node, W_edge, b_edge,
           Wsrc, bsrc, Wdst, bdst, Wm1, bm1, Wm2, bm2, ln_g, ln_b,
           Wh1, bh1, Wh2, bh2, Wg1, bg1, Wg2, bg2):
    f32 = jnp.float32
    hi = jax.lax.Precision.HIGHEST
    Wm1s, Wm1d, Wm1e = Wm1[:, :H, :], Wm1[:, H:2 * H, :], Wm1[:, 2 * H:, :]
    WA = jnp.einsum('lij,ljk->lik', Wsrc, Wm1s, precision=hi)
    cA = jnp.einsum('lj,ljk->lk', bsrc, Wm1s, precision=hi)
    WB = jnp.einsum('lij,ljk->lik', Wdst, Wm1d, precision=hi)
    cB = jnp.einsum('lj,ljk->lk', bdst, Wm1d, precision=hi)
    WC = jnp.einsum('ij,ljk->lik', W_edge, Wm1e, precision=hi)
    cC = jnp.einsum('j,ljk->lk', b_edge, Wm1e, precision=hi) + bm1

    c_layers = [_edgeprep(edge_attr, WC[l], cC[l].reshape(1, H))
                for l in range(L)]
    srcs = edge_index[0].reshape(NW, CPT, K)
    dsts = edge_index[1].reshape(NW, CPT, K)

    h, a, b = _h0(x, W_node, b_node.reshape(1, H),
                  WA[0], cA[0].reshape(1, H), WB[0], cB[0].reshape(1, H))

    deg_raw = None
    for l in range(L):
        if l == 0:
            s_raw, deg_raw = _edge_deg(a, b, c_layers[l], srcs, dsts)
        else:
            s_raw = _edge(a, b, c_layers[l], srcs, dsts)
        if l < L - 1:
            h, a, b = _post_mid(
                h, a, b, s_raw, deg_raw, bm1[l].reshape(1, H), Wm2[l],
                bm2[l].reshape(1, H),
                WA[l + 1], cA[l + 1].reshape(1, H),
                WB[l + 1], cB[l + 1].reshape(1, H))
        else:
            h = _post_last(
                h, a, b, s_raw, deg_raw, bm1[l].reshape(1, H), Wm2[l],
                bm2[l].reshape(1, H))

    wg2p = jnp.concatenate([Wg2, jnp.zeros((32, 3), f32)], axis=1)
    bg2p = jnp.concatenate([bg2, jnp.zeros((3,), f32)]).reshape(1, 8)
    nl, pat = _final(
        h, batch.reshape(N, 1), ln_g.reshape(1, H), ln_b.reshape(1, H),
        Wh1, bh1.reshape(1, 32), Wh2, bh2.reshape(1, 1),
        Wg1, bg1.reshape(1, 32), wg2p, bg2p)
    return nl.reshape(-1), pat[:, :5]


# speedup vs baseline: 12.5695x; 1.5412x over previous
import functools

import jax
import jax.numpy as jnp
from jax import lax
from jax.experimental import pallas as pl
from jax.experimental.pallas import tpu as pltpu
from jax.experimental.pallas import tpu_sc as plsc

N = 10000
E = 320000
NODE_IN = 128
EDGE_IN = 16
H = 64
L = 4
NG = 16

NC = 2
NS = 16
NW = NC * NS
EPW = E // NW
K = 80
CPT = EPW // K
NP = 10240
RPS = NP // NS
ZR = 128
DW = 16

_mesh = plsc.VectorSubcoreMesh(
    core_axis_name="c", subcore_axis_name="s", num_cores=NC, num_subcores=NS)


def _edge_body(with_deg, co, *refs):
    if with_deg:
        (a_hbm, b_hbm, c_hbm, src_hbm, dst_hbm, s_out, d_out,
         sidx, didx, buf_a0, buf_b0, buf_c0, buf_o0,
         buf_a1, buf_b1, buf_c1, buf_o1, ones_b, zb, zb16,
         s_sh, d_sh, sem_a0, sem_b0, sem_c0, sem_a1, sem_b1, sem_c1) = refs
    else:
        (a_hbm, b_hbm, c_hbm, src_hbm, dst_hbm, s_out,
         sidx, didx, buf_a0, buf_b0, buf_c0, buf_o0,
         buf_a1, buf_b1, buf_c1, buf_o1, zb,
         s_sh, sem_a0, sem_b0, sem_c0, sem_a1, sem_b1, sem_c1) = refs
    ci = lax.axis_index("c")
    si = lax.axis_index("s")
    wid = si * NC + ci

    zero16 = jnp.zeros((16,), jnp.float32)
    def zrow(i, _):
        for q in range(H // 16):
            zb[i, pl.ds(q * 16, 16)] = zero16
        return 0
    lax.fori_loop(0, ZR, zrow, 0)
    def zcopy(j, _):
        pltpu.sync_copy(zb, s_sh.at[pl.ds(si * RPS + j * ZR, ZR)])
        return 0
    lax.fori_loop(0, RPS // ZR, zcopy, 0)
    if with_deg:
        def zrow16(i, _):
            zb16[i, :] = zero16
            return 0
        lax.fori_loop(0, ZR, zrow16, 0)
        def zcopy16(j, _):
            pltpu.sync_copy(zb16, d_sh.at[pl.ds(si * RPS + j * ZR, ZR)])
            return 0
        lax.fori_loop(0, RPS // ZR, zcopy16, 0)
        one16 = jnp.ones((16,), jnp.float32)
        def orow(i, _):
            ones_b[i, :] = one16
            return 0
        lax.fori_loop(0, K, orow, 0)
    plsc.subcore_barrier()

    def c_view(c):
        return c_hbm.at[pl.ds(wid * EPW + c * K, K), pl.ds(co, H)]

    def issue(c, buf_a, buf_b, buf_c, sem_a, sem_b, sem_c):
        pltpu.async_copy(a_hbm.at[sidx.at[c]], buf_a, sem_a)
        pltpu.async_copy(b_hbm.at[didx.at[c]], buf_b, sem_b)
        pltpu.async_copy(c_view(c), buf_c, sem_c)

    def half(c, buf_a, buf_b, buf_c, buf_o, sem_a, sem_b, sem_c):
        pltpu.make_async_copy(a_hbm.at[sidx.at[c]], buf_a, sem_a).wait()
        pltpu.make_async_copy(b_hbm.at[didx.at[c]], buf_b, sem_b).wait()
        pltpu.make_async_copy(c_view(c), buf_c, sem_c).wait()
        def row(r, _):
            for q in range(H // 16):
                sl = pl.ds(q * 16, 16)
                buf_o[r, sl] = jnp.maximum(
                    buf_a[r, sl] + buf_b[r, sl] + buf_c[r, sl], 0.0)
            return 0
        lax.fori_loop(0, K, row, 0)
        @pl.when(c + 2 < CPT)
        def _():
            issue(c + 2, buf_a, buf_b, buf_c, sem_a, sem_b, sem_c)
        pltpu.sync_copy(buf_o, s_sh.at[didx.at[c]], add=True)
        if with_deg:
            pltpu.sync_copy(ones_b, d_sh.at[didx.at[c]], add=True)

    pltpu.sync_copy(src_hbm.at[wid], sidx)
    pltpu.sync_copy(dst_hbm.at[wid], didx)
    issue(0, buf_a0, buf_b0, buf_c0, sem_a0, sem_b0, sem_c0)
    issue(1, buf_a1, buf_b1, buf_c1, sem_a1, sem_b1, sem_c1)

    def pair(j, _):
        half(2 * j, buf_a0, buf_b0, buf_c0, buf_o0, sem_a0, sem_b0, sem_c0)
        half(2 * j + 1, buf_a1, buf_b1, buf_c1, buf_o1,
             sem_a1, sem_b1, sem_c1)
        return 0
    lax.fori_loop(0, (CPT - 1) // 2, pair, 0)
    half(CPT - 1, buf_a0, buf_b0, buf_c0, buf_o0, sem_a0, sem_b0, sem_c0)

    plsc.subcore_barrier()
    def wcopy(j, _):
        rows = pl.ds(si * RPS + j * ZR, ZR)
        pltpu.sync_copy(s_sh.at[rows], s_out.at[ci, rows])
        if with_deg:
            pltpu.sync_copy(d_sh.at[rows], d_out.at[ci, rows])
        return 0
    lax.fori_loop(0, RPS // ZR, wcopy, 0)


def _make_edge_kernel(with_deg, co):
    out_type = [jax.ShapeDtypeStruct((NC, NP, H), jnp.float32)]
    scratch = [
        pltpu.VMEM((CPT, K), jnp.int32),
        pltpu.VMEM((CPT, K), jnp.int32),
    ]
    scratch += [pltpu.VMEM((K, H), jnp.float32)] * 8
    if with_deg:
        out_type.append(jax.ShapeDtypeStruct((NC, NP, DW), jnp.float32))
        scratch.append(pltpu.VMEM((K, DW), jnp.float32))
    scratch.append(pltpu.VMEM((ZR, H), jnp.float32))
    if with_deg:
        scratch.append(pltpu.VMEM((ZR, DW), jnp.float32))
    scratch.append(pltpu.VMEM_SHARED((NP, H), jnp.float32))
    if with_deg:
        scratch.append(pltpu.VMEM_SHARED((NP, DW), jnp.float32))
    scratch += [pltpu.SemaphoreType.DMA] * 6
    return pl.kernel(
        functools.partial(_edge_body, with_deg, co),
        out_type=tuple(out_type) if with_deg else out_type[0],
        mesh=_mesh,
        scratch_types=scratch,
        compiler_params=pltpu.CompilerParams(use_tc_tiling_on_sc=False),
    )


_edge_kernels = [_make_edge_kernel(l == 0, (l % 2) * H) for l in range(L)]


EB = 8000


def _edgeprep_body(ea_ref, wc_ref, cc_ref, out_ref):
    out_ref[...] = (jnp.dot(ea_ref[...], wc_ref[...],
                            preferred_element_type=jnp.float32) + cc_ref[...])


_edgeprep = pl.pallas_call(
    _edgeprep_body,
    grid=(E // EB,),
    in_specs=[
        pl.BlockSpec((EB, EDGE_IN), lambda i: (i, 0)),
        pl.BlockSpec((EDGE_IN, 2 * H), lambda i: (0, 0)),
        pl.BlockSpec((1, 2 * H), lambda i: (0, 0)),
    ],
    out_specs=pl.BlockSpec((EB, 2 * H), lambda i: (i, 0)),
    out_shape=jax.ShapeDtypeStruct((E, 2 * H), jnp.float32),
)


def _h0_body(x_ref, wn_ref, bn_ref, wa_ref, ca_ref, wb_ref, cb_ref,
             h_ref, a_ref, b_ref):
    h = jnp.dot(x_ref[...], wn_ref[...],
                preferred_element_type=jnp.float32) + bn_ref[...]
    h_ref[...] = h
    a_ref[...] = jnp.dot(h, wa_ref[...],
                         preferred_element_type=jnp.float32) + ca_ref[...]
    b_ref[...] = jnp.dot(h, wb_ref[...],
                         preferred_element_type=jnp.float32) + cb_ref[...]


_h0 = pl.pallas_call(
    _h0_body,
    out_shape=(jax.ShapeDtypeStruct((N, H), jnp.float32),
               jax.ShapeDtypeStruct((N, H), jnp.float32),
               jax.ShapeDtypeStruct((N, H), jnp.float32)),
)


def _post_body(last, *refs):
    if last:
        (h_ref, a_ref, b_ref, s_ref, deg_ref, bm1_ref, wm2_ref, bm2_ref,
         hn_ref) = refs
    else:
        (h_ref, a_ref, b_ref, s_ref, deg_ref, bm1_ref, wm2_ref, bm2_ref,
         wa_ref, ca_ref, wb_ref, cb_ref, hn_ref, an_ref, bn_ref) = refs
    s = s_ref[0, :N, :] + s_ref[1, :N, :] + jnp.maximum(
        a_ref[...] + b_ref[...] + bm1_ref[...], 0.0)
    degf = deg_ref[0, :N, 0:1] + deg_ref[1, :N, 0:1] + 1.0
    aggr = jnp.dot(s, wm2_ref[...],
                   preferred_element_type=jnp.float32) + degf * bm2_ref[...]
    hn = jnp.maximum(h_ref[...] + aggr, 0.0)
    hn_ref[...] = hn
    if not last:
        an_ref[...] = jnp.dot(hn, wa_ref[...],
                              preferred_element_type=jnp.float32) + ca_ref[...]
        bn_ref[...] = jnp.dot(hn, wb_ref[...],
                              preferred_element_type=jnp.float32) + cb_ref[...]


_post_mid = pl.pallas_call(
    functools.partial(_post_body, False),
    out_shape=(jax.ShapeDtypeStruct((N, H), jnp.float32),
               jax.ShapeDtypeStruct((N, H), jnp.float32),
               jax.ShapeDtypeStruct((N, H), jnp.float32)),
)

_post_last = pl.pallas_call(
    functools.partial(_post_body, True),
    out_shape=jax.ShapeDtypeStruct((N, H), jnp.float32),
)


def _final_body(h_ref, batch_ref, lng_ref, lnb_ref, wh1_ref, bh1_ref,
                wh2_ref, bh2_ref, wg1_ref, bg1_ref, wg2_ref, bg2_ref,
                nl_ref, pat_ref):
    h = h_ref[...]
    mu = jnp.mean(h, axis=1, keepdims=True)
    var = jnp.mean((h - mu) ** 2, axis=1, keepdims=True)
    xn = (h - mu) * lax.rsqrt(var + 1e-5) * lng_ref[...] + lnb_ref[...]
    t = jnp.maximum(
        jnp.dot(xn, wh1_ref[...], preferred_element_type=jnp.float32)
        + bh1_ref[...], 0.0)
    logit = jnp.dot(t, wh2_ref[...],
                    preferred_element_type=jnp.float32) + bh2_ref[...]
    nl_ref[...] = jax.nn.sigmoid(logit)
    m = (batch_ref[...] == lax.broadcasted_iota(jnp.int32, (1, NG), 1)
         ).astype(jnp.float32)
    sums = lax.dot_general(m, h, (((0,), (0,)), ((), ())),
                           preferred_element_type=jnp.float32)
    counts = lax.dot_general(m, jnp.ones((N, 1), jnp.float32),
                             (((0,), (0,)), ((), ())),
                             preferred_element_type=jnp.float32)
    g = sums / jnp.maximum(counts, 1.0)
    gt = jnp.maximum(
        jnp.dot(g, wg1_ref[...], preferred_element_type=jnp.float32)
        + bg1_ref[...], 0.0)
    pat_ref[...] = jnp.dot(gt, wg2_ref[...],
                           preferred_element_type=jnp.float32) + bg2_ref[...]


_final = pl.pallas_call(
    _final_body,
    out_shape=(jax.ShapeDtypeStruct((N, 1), jnp.float32),
               jax.ShapeDtypeStruct((NG, 8), jnp.float32)),
)


def kernel(x, edge_index, edge_attr, batch, W_node, b_node, W_edge, b_edge,
           Wsrc, bsrc, Wdst, bdst, Wm1, bm1, Wm2, bm2, ln_g, ln_b,
           Wh1, bh1, Wh2, bh2, Wg1, bg1, Wg2, bg2):
    f32 = jnp.float32
    hi = jax.lax.Precision.HIGHEST
    Wm1s, Wm1d, Wm1e = Wm1[:, :H, :], Wm1[:, H:2 * H, :], Wm1[:, 2 * H:, :]
    WA = jnp.einsum('lij,ljk->lik', Wsrc, Wm1s, precision=hi)
    cA = jnp.einsum('lj,ljk->lk', bsrc, Wm1s, precision=hi)
    WB = jnp.einsum('lij,ljk->lik', Wdst, Wm1d, precision=hi)
    cB = jnp.einsum('lj,ljk->lk', bdst, Wm1d, precision=hi)
    WC = jnp.einsum('ij,ljk->lik', W_edge, Wm1e, precision=hi)
    cC = jnp.einsum('j,ljk->lk', b_edge, Wm1e, precision=hi) + bm1

    c_pairs = [
        _edgeprep(edge_attr,
                  jnp.concatenate([WC[p], WC[p + 1]], axis=1),
                  jnp.concatenate([cC[p], cC[p + 1]]).reshape(1, 2 * H))
        for p in (0, 2)]
    srcs = edge_index[0].reshape(NW, CPT, K)
    dsts = edge_index[1].reshape(NW, CPT, K)

    h, a, b = _h0(x, W_node, b_node.reshape(1, H),
                  WA[0], cA[0].reshape(1, H), WB[0], cB[0].reshape(1, H))

    deg_raw = None
    for l in range(L):
        if l == 0:
            s_raw, deg_raw = _edge_kernels[l](a, b, c_pairs[l // 2], srcs, dsts)
        else:
            s_raw = _edge_kernels[l](a, b, c_pairs[l // 2], srcs, dsts)
        if l < L - 1:
            h, a, b = _post_mid(
                h, a, b, s_raw, deg_raw, bm1[l].reshape(1, H), Wm2[l],
                bm2[l].reshape(1, H),
                WA[l + 1], cA[l + 1].reshape(1, H),
                WB[l + 1], cB[l + 1].reshape(1, H))
        else:
            h = _post_last(
                h, a, b, s_raw, deg_raw, bm1[l].reshape(1, H), Wm2[l],
                bm2[l].reshape(1, H))

    wg2p = jnp.concatenate([Wg2, jnp.zeros((32, 3), f32)], axis=1)
    bg2p = jnp.concatenate([bg2, jnp.zeros((3,), f32)]).reshape(1, 8)
    nl, pat = _final(
        h, batch.reshape(N, 1), ln_g.reshape(1, H), ln_b.reshape(1, H),
        Wh1, bh1.reshape(1, 32), Wh2, bh2.reshape(1, 1),
        Wg1, bg1.reshape(1, 32), wg2p, bg2p)
    return nl.reshape(-1), pat[:, :5]
